# baseline (speedup 1.0000x reference)
import functools

import jax
import jax.numpy as jnp
from jax import lax
from jax.experimental import pallas as pl
from jax.experimental.pallas import tpu as pltpu
from jax.experimental.pallas import tpu_sc as plsc

N = 10000
E = 320000
D = 128
DE = 16
H = 64
R = 5

NC, NS = 2, 16
NW = NC * NS
E1 = 102400
E2 = E - E1
GC = 128
OC = 1280

EC = E // NS
HALF = EC // NC
NR = (N + 127) // 128
NPAD = NR * 128

EB = 6400


def _ab_body(x_ref, wa_ref, wb_ref, a_ref, b_ref):
    xb = x_ref[...]
    a_ref[...] = lax.dot(xb, wa_ref[...], preferred_element_type=jnp.float32)
    b_ref[...] = lax.dot(xb, wb_ref[...], preferred_element_type=jnp.float32)


def _score_body(g_ref, ea_ref, wc_ref, b1_ref, w2_ref, b2_ref, p_ref):
    c = lax.dot_general(ea_ref[...], wc_ref[...], (((0,), (0,)), ((), ())),
                        preferred_element_type=jnp.float32)
    h = g_ref[:, :H] + c + b1_ref[...]
    h = jnp.maximum(h, 0.0)
    s = lax.dot(h, w2_ref[...], preferred_element_type=jnp.float32) + b2_ref[...]
    p_ref[...] = jax.nn.sigmoid(s)


_sc_mesh = plsc.VectorSubcoreMesh(core_axis_name="c", subcore_axis_name="s")
_sc_params = pltpu.CompilerParams(use_tc_tiling_on_sc=False,
                                  needs_layout_passes=False)


def _make_gather(edge_base, n_edges):
    ew = n_edges // NW
    noc = -(-ew // OC)
    ewp = noc * OC

    @functools.partial(
        pl.kernel, mesh=_sc_mesh, compiler_params=_sc_params,
        out_type=jax.ShapeDtypeStruct((n_edges, 2 * H), jnp.float32),
        scratch_types=[
            pltpu.VMEM((ewp,), jnp.int32),
            pltpu.VMEM((ewp,), jnp.int32),
            pltpu.VMEM((OC, H), jnp.float32),
            pltpu.SemaphoreType.DMA,
        ],
    )
    def _gather_ab(a_hbm, b_hbm, src_hbm, dst_hbm, g_hbm,
                   idxs_v, idxd_v, rows_v, sem):
        wid = lax.axis_index("s") * NC + lax.axis_index("c")
        obase = jnp.minimum(wid * ew, n_edges - ewp)
        base = edge_base + obase
        pltpu.sync_copy(src_hbm.at[pl.ds(base, ewp)], idxs_v)
        pltpu.sync_copy(dst_hbm.at[pl.ds(base, ewp)], idxd_v)
        for oc in range(noc):
            descs = [
                pltpu.async_copy(
                    a_hbm.at[idxs_v.at[pl.ds(oc * OC + j * GC, GC)]],
                    rows_v.at[pl.ds(j * GC, GC)], sem)
                for j in range(OC // GC)
            ]
            for d in descs:
                d.wait()
            descs = [
                pltpu.async_copy(
                    b_hbm.at[idxd_v.at[pl.ds(oc * OC + j * GC, GC)]],
                    rows_v.at[pl.ds(j * GC, GC)], sem, add=True)
                for j in range(OC // GC)
            ]
            for d in descs:
                d.wait()
            pltpu.sync_copy(rows_v,
                            g_hbm.at[pl.ds(obase + oc * OC, OC), pl.ds(0, H)])
    return _gather_ab


_gather_h1 = _make_gather(0, E1)
_gather_h2 = _make_gather(E1, E2)


@functools.partial(
    pl.kernel, mesh=_sc_mesh, compiler_params=_sc_params,
    out_type=jax.ShapeDtypeStruct((R, E), jnp.float32),
    scratch_types=[
        pltpu.VMEM((EC,), jnp.int32),
        pltpu.VMEM((EC,), jnp.int32),
        pltpu.VMEM((EC,), jnp.float32),
        pltpu.VMEM((EC,), jnp.float32),
        pltpu.VMEM((NR, 128), jnp.float32),
        pltpu.VMEM((NR, 128), jnp.float32),
        pltpu.VMEM((NR,), jnp.int32),
        pltpu.VMEM_SHARED((NR, 128), jnp.float32),
    ],
)
def _frontier(src_hbm, dst_hbm, p_hbm, m0_hbm, rows_hbm, out_hbm,
              srcv, dstv, pv, pmv, mloc, updv, rowids, sm):
    cid = lax.axis_index("c")
    sid = lax.axis_index("s")
    tbase = sid * EC
    pltpu.sync_copy(src_hbm.at[pl.ds(tbase, EC)], srcv)
    pltpu.sync_copy(dst_hbm.at[pl.ds(tbase, EC)], dstv)
    pltpu.sync_copy(p_hbm.at[pl.ds(tbase, EC)], pv)

    pltpu.sync_copy(m0_hbm, mloc)
    pltpu.sync_copy(rows_hbm, rowids)

    @pl.when(sid == 0)
    def _():
        pltpu.sync_copy(m0_hbm, sm)

    plsc.subcore_barrier()

    zeros = jnp.zeros((16,), jnp.float32)
    ones = jnp.ones((16,), jnp.float32)

    def zbody(i, _):
        updv[i >> 3, pl.ds((i & 7) * 16, 16)] = zeros
        return 0
    lax.fori_loop(0, NR * 8, zbody, 0)

    U = 10
    for r in range(R):
        def ebody(k, _):
            offs = [k * (U * 16) + u * 16 for u in range(U)]
            svs = [srcv[pl.ds(o, 16)] for o in offs]
            dvs = [dstv[pl.ds(o, 16)] for o in offs]
            pps = [pv[pl.ds(o, 16)] for o in offs]
            mss = [plsc.load_gather(mloc, [sv >> 7, sv & 127]) for sv in svs]
            mds = [plsc.load_gather(mloc, [dv >> 7, dv & 127]) for dv in dvs]
            pms = [jnp.where((ms > 0.5) & (md < 0.5), pp, 0.0)
                   for ms, md, pp in zip(mss, mds, pps)]
            for u in range(U):
                pmv[pl.ds(offs[u], 16)] = pms[u]
            for u in range(U):
                plsc.store_scatter(updv, [dvs[u] >> 7, dvs[u] & 127], ones,
                                   mask=pms[u] > 0.5)
            return 0
        lax.fori_loop(0, EC // (U * 16), ebody, 0)

        pltpu.sync_copy(pmv.at[pl.ds(cid * HALF, HALF)],
                        out_hbm.at[r, pl.ds(tbase + cid * HALF, HALF)])

        if r < R - 1:
            pltpu.sync_copy(updv, sm.at[rowids], add=True)
            plsc.subcore_barrier()
            pltpu.sync_copy(sm, mloc)
            plsc.subcore_barrier()


def kernel(x, edge_index, edge_attr, W1, b1, W2, b2):
    src = edge_index[0]
    dst = edge_index[1]
    w1a = W1[:D]
    w1b = W1[D:2 * D]
    w1c = W1[2 * D:]

    a_tab, b_tab = pl.pallas_call(
        _ab_body,
        grid=(N // 1000,),
        in_specs=[
            pl.BlockSpec((1000, D), lambda i: (i, 0)),
            pl.BlockSpec((D, H), lambda i: (0, 0)),
            pl.BlockSpec((D, H), lambda i: (0, 0)),
        ],
        out_specs=[
            pl.BlockSpec((1000, H), lambda i: (i, 0)),
            pl.BlockSpec((1000, H), lambda i: (i, 0)),
        ],
        out_shape=[
            jax.ShapeDtypeStruct((N, H), jnp.float32),
            jax.ShapeDtypeStruct((N, H), jnp.float32),
        ],
    )(x, w1a, w1b)

    g1 = _gather_h1(a_tab, b_tab, src, dst)
    g2 = _gather_h2(a_tab, b_tab, src, dst)

    def _score(g, n_edges, ea_block_off):
        return pl.pallas_call(
            _score_body,
            grid=(n_edges // EB,),
            in_specs=[
                pl.BlockSpec((EB, 2 * H), lambda i: (i, 0)),
                pl.BlockSpec((DE, EB), lambda i: (0, i + ea_block_off)),
                pl.BlockSpec((DE, H), lambda i: (0, 0)),
                pl.BlockSpec((1, H), lambda i: (0, 0)),
                pl.BlockSpec((H, 1), lambda i: (0, 0)),
                pl.BlockSpec((1, 1), lambda i: (0, 0)),
            ],
            out_specs=pl.BlockSpec((EB, 1), lambda i: (i, 0)),
            out_shape=jax.ShapeDtypeStruct((n_edges, 1), jnp.float32),
        )(g, edge_attr.T, w1c, b1.reshape(1, H), W2, b2.reshape(1, 1))

    p1 = _score(g1, E1, 0).reshape(E1)
    p2 = _score(g2, E2, E1 // EB).reshape(E2)
    p = jnp.concatenate([p1, p2])

    m0 = jnp.where(jnp.arange(NPAD) % 10 == 0, 1.0, 0.0)
    m0 = m0.astype(jnp.float32).reshape(NR, 128)
    rowids = jnp.arange(NR, dtype=jnp.int32)
    return _frontier(src, dst, p, m0, rowids)


# speedup vs baseline: 1.0439x; 1.0439x over previous
import functools

import jax
import jax.numpy as jnp
from jax import lax
from jax.experimental import pallas as pl
from jax.experimental.pallas import tpu as pltpu
from jax.experimental.pallas import tpu_sc as plsc

N = 10000
E = 320000
D = 128
DE = 16
H = 64
R = 5

NC, NS = 2, 16
NW = NC * NS
E1 = E // 2
E2 = E - E1
GC = 128
OC = 1280

EC = E // NS
HALF = EC // NC
NR = (N + 127) // 128
NPAD = NR * 128

EB = 6400


def _ab_body(x_ref, wa_ref, wb_ref, a_ref, b_ref):
    xb = x_ref[...]
    a_ref[...] = lax.dot(xb, wa_ref[...], preferred_element_type=jnp.float32)
    b_ref[...] = lax.dot(xb, wb_ref[...], preferred_element_type=jnp.float32)


def _score_body(g_ref, ea_ref, wc_ref, b1_ref, w2_ref, b2_ref, p_ref):
    c = lax.dot_general(ea_ref[...], wc_ref[...], (((0,), (0,)), ((), ())),
                        preferred_element_type=jnp.float32)
    h = g_ref[:, :H] + c + b1_ref[...]
    h = jnp.maximum(h, 0.0)
    s = lax.dot(h, w2_ref[...], preferred_element_type=jnp.float32) + b2_ref[...]
    p_ref[...] = jax.nn.sigmoid(s)


_sc_mesh = plsc.VectorSubcoreMesh(core_axis_name="c", subcore_axis_name="s")
_sc_params = pltpu.CompilerParams(use_tc_tiling_on_sc=False,
                                  needs_layout_passes=False)


def _make_gather(edge_base, n_edges):
    ew = n_edges // NW
    noc = -(-ew // OC)
    ewp = noc * OC

    @functools.partial(
        pl.kernel, mesh=_sc_mesh, compiler_params=_sc_params,
        out_type=jax.ShapeDtypeStruct((n_edges, 2 * H), jnp.float32),
        scratch_types=[
            pltpu.VMEM((ewp,), jnp.int32),
            pltpu.VMEM((ewp,), jnp.int32),
            pltpu.VMEM((OC, H), jnp.float32),
            pltpu.SemaphoreType.DMA,
        ],
    )
    def _gather_ab(a_hbm, b_hbm, src_hbm, dst_hbm, g_hbm,
                   idxs_v, idxd_v, rows_v, sem):
        wid = lax.axis_index("s") * NC + lax.axis_index("c")
        obase = jnp.minimum(wid * ew, n_edges - ewp)
        base = edge_base + obase
        pltpu.sync_copy(src_hbm.at[pl.ds(base, ewp)], idxs_v)
        pltpu.sync_copy(dst_hbm.at[pl.ds(base, ewp)], idxd_v)
        for oc in range(noc):
            descs = [
                pltpu.async_copy(
                    a_hbm.at[idxs_v.at[pl.ds(oc * OC + j * GC, GC)]],
                    rows_v.at[pl.ds(j * GC, GC)], sem)
                for j in range(OC // GC)
            ]
            for d in descs:
                d.wait()
            descs = [
                pltpu.async_copy(
                    b_hbm.at[idxd_v.at[pl.ds(oc * OC + j * GC, GC)]],
                    rows_v.at[pl.ds(j * GC, GC)], sem, add=True)
                for j in range(OC // GC)
            ]
            for d in descs:
                d.wait()
            pltpu.sync_copy(rows_v,
                            g_hbm.at[pl.ds(obase + oc * OC, OC), pl.ds(0, H)])
    return _gather_ab


_gather_h1 = _make_gather(0, E1)
_gather_h2 = _make_gather(E1, E2)


@functools.partial(
    pl.kernel, mesh=_sc_mesh, compiler_params=_sc_params,
    out_type=jax.ShapeDtypeStruct((R, E), jnp.float32),
    scratch_types=[
        pltpu.VMEM((EC,), jnp.int32),
        pltpu.VMEM((EC,), jnp.int32),
        pltpu.VMEM((EC,), jnp.float32),
        pltpu.VMEM((EC,), jnp.float32),
        pltpu.VMEM((NR, 128), jnp.float32),
        pltpu.VMEM((NR, 128), jnp.float32),
        pltpu.VMEM((NR,), jnp.int32),
        pltpu.VMEM_SHARED((NR, 128), jnp.float32),
    ],
)
def _frontier(src_hbm, dst_hbm, p_hbm, m0_hbm, rows_hbm, out_hbm,
              srcv, dstv, pv, pmv, mloc, updv, rowids, sm):
    cid = lax.axis_index("c")
    sid = lax.axis_index("s")
    tbase = sid * EC
    pltpu.sync_copy(src_hbm.at[pl.ds(tbase, EC)], srcv)
    pltpu.sync_copy(dst_hbm.at[pl.ds(tbase, EC)], dstv)
    pltpu.sync_copy(p_hbm.at[pl.ds(tbase, EC)], pv)

    pltpu.sync_copy(m0_hbm, mloc)
    pltpu.sync_copy(rows_hbm, rowids)

    @pl.when(sid == 0)
    def _():
        pltpu.sync_copy(m0_hbm, sm)

    plsc.subcore_barrier()

    zeros = jnp.zeros((16,), jnp.float32)
    ones = jnp.ones((16,), jnp.float32)

    def zbody(i, _):
        updv[i >> 3, pl.ds((i & 7) * 16, 16)] = zeros
        return 0
    lax.fori_loop(0, NR * 8, zbody, 0)

    U = 10
    for r in range(R):
        def ebody(k, _):
            offs = [k * (U * 16) + u * 16 for u in range(U)]
            svs = [srcv[pl.ds(o, 16)] for o in offs]
            dvs = [dstv[pl.ds(o, 16)] for o in offs]
            pps = [pv[pl.ds(o, 16)] for o in offs]
            mss = [plsc.load_gather(mloc, [sv >> 7, sv & 127]) for sv in svs]
            mds = [plsc.load_gather(mloc, [dv >> 7, dv & 127]) for dv in dvs]
            pms = [jnp.where((ms > 0.5) & (md < 0.5), pp, 0.0)
                   for ms, md, pp in zip(mss, mds, pps)]
            for u in range(U):
                pmv[pl.ds(offs[u], 16)] = pms[u]
            for u in range(U):
                plsc.store_scatter(updv, [dvs[u] >> 7, dvs[u] & 127], ones,
                                   mask=pms[u] > 0.5)
            return 0
        lax.fori_loop(0, EC // (U * 16), ebody, 0)

        pltpu.sync_copy(pmv.at[pl.ds(cid * HALF, HALF)],
                        out_hbm.at[r, pl.ds(tbase + cid * HALF, HALF)])

        if r < R - 1:
            pltpu.sync_copy(updv, sm.at[rowids], add=True)
            plsc.subcore_barrier()
            pltpu.sync_copy(sm, mloc)
            plsc.subcore_barrier()


def kernel(x, edge_index, edge_attr, W1, b1, W2, b2):
    src = edge_index[0]
    dst = edge_index[1]
    w1a = W1[:D]
    w1b = W1[D:2 * D]
    w1c = W1[2 * D:]

    a_tab, b_tab = pl.pallas_call(
        _ab_body,
        grid=(N // 1000,),
        in_specs=[
            pl.BlockSpec((1000, D), lambda i: (i, 0)),
            pl.BlockSpec((D, H), lambda i: (0, 0)),
            pl.BlockSpec((D, H), lambda i: (0, 0)),
        ],
        out_specs=[
            pl.BlockSpec((1000, H), lambda i: (i, 0)),
            pl.BlockSpec((1000, H), lambda i: (i, 0)),
        ],
        out_shape=[
            jax.ShapeDtypeStruct((N, H), jnp.float32),
            jax.ShapeDtypeStruct((N, H), jnp.float32),
        ],
    )(x, w1a, w1b)

    g1 = _gather_h1(a_tab, b_tab, src, dst)
    g2 = _gather_h2(a_tab, b_tab, src, dst)

    def _score(g, n_edges, ea_block_off):
        return pl.pallas_call(
            _score_body,
            grid=(n_edges // EB,),
            in_specs=[
                pl.BlockSpec((EB, 2 * H), lambda i: (i, 0)),
                pl.BlockSpec((DE, EB), lambda i: (0, i + ea_block_off)),
                pl.BlockSpec((DE, H), lambda i: (0, 0)),
                pl.BlockSpec((1, H), lambda i: (0, 0)),
                pl.BlockSpec((H, 1), lambda i: (0, 0)),
                pl.BlockSpec((1, 1), lambda i: (0, 0)),
            ],
            out_specs=pl.BlockSpec((EB, 1), lambda i: (i, 0)),
            out_shape=jax.ShapeDtypeStruct((n_edges, 1), jnp.float32),
        )(g, edge_attr.T, w1c, b1.reshape(1, H), W2, b2.reshape(1, 1))

    p1 = _score(g1, E1, 0).reshape(E1)
    p2 = _score(g2, E2, E1 // EB).reshape(E2)
    p = jnp.concatenate([p1, p2])

    m0 = jnp.where(jnp.arange(NPAD) % 10 == 0, 1.0, 0.0)
    m0 = m0.astype(jnp.float32).reshape(NR, 128)
    rowids = jnp.arange(NR, dtype=jnp.int32)
    return _frontier(src, dst, p, m0, rowids)
